# baseline (speedup 1.0000x reference)
import functools

import jax
import jax.numpy as jnp
from jax import lax
from jax.experimental import pallas as pl
from jax.experimental.pallas import tpu as pltpu
from jax.experimental.pallas import tpu_sc as plsc

N = 10000
E = 320000
H = 64
W128 = 128
OUT = 128
B = 16

NCORE = 2
NSUB = 16
NW = NCORE * NSUB

EBLK = 3200
NBLK = 2000

GCH = 128
NCHUNKS = E // GCH

SEG = 320
DCH = 2560
NDCH = E // DCH
IDXCAP = DCH + 16


def _mesh():
    return plsc.VectorSubcoreMesh(core_axis_name="c", subcore_axis_name="s")


def _wid():
    return lax.axis_index("s") * NCORE + lax.axis_index("c")


def _gather_body(t_hbm, src_hbm, dst_hbm, m_hbm,
                 sbuf0, sbuf1, dbuf0, dbuf1, abuf0, abuf1, cbuf0, cbuf1,
                 mbuf, sem_a0, sem_a1, sem_c0, sem_c1):
    w = _wid()
    nch = (NCHUNKS - w + NW - 1) // NW
    sbuf = (sbuf0, sbuf1)
    dbuf = (dbuf0, dbuf1)
    abuf = (abuf0, abuf1)
    cbuf = (cbuf0, cbuf1)
    sem_a = (sem_a0, sem_a1)
    sem_c = (sem_c0, sem_c1)

    zeros = jnp.zeros((16,), jnp.float32)

    def zrow(r, _):
        for j in range(H, W128, 16):
            mbuf[r, pl.ds(j, 16)] = zeros
        return 0

    lax.fori_loop(0, GCH, zrow, 0)

    def issue(k, b):
        base = (w + k * NW) * GCH
        pltpu.sync_copy(src_hbm.at[pl.ds(base, GCH)], sbuf[b])
        pltpu.sync_copy(dst_hbm.at[pl.ds(base, GCH)], dbuf[b])
        pltpu.async_copy(t_hbm.at[sbuf[b]], abuf[b], sem_a[b])
        pltpu.async_copy(t_hbm.at[dbuf[b]], cbuf[b], sem_c[b])

    def consume(k, b):
        pltpu.make_async_copy(t_hbm.at[sbuf[b]], abuf[b], sem_a[b]).wait()
        pltpu.make_async_copy(t_hbm.at[dbuf[b]], cbuf[b], sem_c[b]).wait()

        def row(r, _):
            for j in range(0, H, 16):
                mbuf[r, pl.ds(j, 16)] = (abuf[b][r, pl.ds(j, 16)]
                                         - cbuf[b][r, pl.ds(H + j, 16)])
            return 0

        lax.fori_loop(0, GCH, row, 0, unroll=2)
        pltpu.sync_copy(mbuf, m_hbm.at[pl.ds((w + k * NW) * GCH, GCH)])

    issue(0, 0)

    def outer(k2, _):
        for b in range(2):
            k = k2 * 2 + b

            @pl.when(k + 1 < nch)
            def _():
                issue(k + 1, 1 - b)

            @pl.when(k < nch)
            def _():
                consume(k, b)
        return 0

    lax.fori_loop(0, (NCHUNKS // NW + 2) // 2, outer, 0)


def _sc_gather(T, src, dst):
    return pl.kernel(
        _gather_body,
        mesh=_mesh(),
        out_type=jax.ShapeDtypeStruct((E, W128), jnp.float32),
        scratch_types=[
            pltpu.VMEM((GCH,), jnp.int32),
            pltpu.VMEM((GCH,), jnp.int32),
            pltpu.VMEM((GCH,), jnp.int32),
            pltpu.VMEM((GCH,), jnp.int32),
            pltpu.VMEM((GCH, W128), jnp.float32),
            pltpu.VMEM((GCH, W128), jnp.float32),
            pltpu.VMEM((GCH, W128), jnp.float32),
            pltpu.VMEM((GCH, W128), jnp.float32),
            pltpu.VMEM((GCH, W128), jnp.float32),
            pltpu.SemaphoreType.DMA,
            pltpu.SemaphoreType.DMA,
            pltpu.SemaphoreType.DMA,
            pltpu.SemaphoreType.DMA,
        ],
    )(T, src, dst)


def _scatter_body(p_hbm, dst_hbm, o_hbm, dbuf0, dbuf1,
                  ibuf0, ibuf1, lbuf0, lbuf1, prow0, prow1, acc,
                  sem0, sem1, dsem0, dsem1):
    w = _wid()
    lo = w * SEG
    hi = lo + SEG
    dbuf = (dbuf0, dbuf1)
    ibuf = (ibuf0, ibuf1)
    lbuf = (lbuf0, lbuf1)
    prow = (prow0, prow1)
    sem = (sem0, sem1)
    dsem = (dsem0, dsem1)

    def prefetch_dst(ci, b):
        pltpu.async_copy(dst_hbm.at[pl.ds(ci * DCH, DCH)], dbuf[b], dsem[b])

    def wait_dst(ci, b):
        pltpu.make_async_copy(dst_hbm.at[pl.ds(ci * DCH, DCH)],
                              dbuf[b], dsem[b]).wait()

    zeros = jnp.zeros((16,), jnp.float32)

    def zrow(r, _):
        for j in range(0, W128, 16):
            acc[r, pl.ds(j, 16)] = zeros
        return 0

    lax.fori_loop(0, SEG, zrow, 0)

    zidx = jnp.zeros((16,), jnp.int32)

    def zi(g, _):
        ibuf0[pl.ds(g * 16, 16)] = zidx
        ibuf1[pl.ds(g * 16, 16)] = zidx
        return 0

    lax.fori_loop(0, IDXCAP // 16, zi, 0)

    def issue_first(b):
        pltpu.async_copy(p_hbm.at[ibuf[b].at[pl.ds(0, GCH)]], prow[b], sem[b])

    def drain(b, cnt):
        pltpu.make_async_copy(p_hbm.at[ibuf[b].at[pl.ds(0, GCH)]],
                              prow[b], sem[b]).wait()

        def sub(j, _):
            @pl.when(j > 0)
            def _():
                pltpu.async_copy(p_hbm.at[ibuf[b].at[pl.ds(j * GCH, GCH)]],
                                 prow[b], sem[b]).wait()

            rows = jnp.minimum(cnt - j * GCH, GCH)

            def rmw(r, _):
                l = lbuf[b][pl.ds(j * GCH + r, 16)][0]
                for jj in range(0, H, 16):
                    sl = pl.ds(jj, 16)
                    acc[l, sl] = jnp.maximum(acc[l, sl], prow[b][r, sl])
                return 0

            lax.fori_loop(0, rows, rmw, 0)
            return 0

        lax.fori_loop(0, (cnt + GCH - 1) // GCH, sub, 0)

    iota = lax.iota(jnp.int32, 16)

    def _lane_gather(v, idx):
        return v.at[idx].get(mode="promise_in_bounds")

    stepm = {s: ((iota - s) >> 31) + 1 for s in (1, 2, 4, 8)}

    def grp(ci, g, cnt, b):
        d = dbuf[b][pl.ds(g * 16, 16)]
        u = d - lo
        mi = ((u >> 31) | ((SEG - 1 - u) >> 31)) + 1
        pc = mi
        for s in (1, 2, 4, 8):
            sh = _lane_gather(pc, jnp.maximum(iota - s, 0))
            pc = pc + sh * stepm[s]
        total = pc[15]

        @pl.when(total > 0)
        def _():
            inv = jnp.zeros((16,), jnp.int32)
            for s in (8, 4, 2, 1):
                probe = _lane_gather(pc, jnp.minimum(inv + (s - 1), 15))
                inv = inv + s * (((iota - probe) >> 31) + 1)
            inv = jnp.minimum(inv, 15)
            packed = (((ci * DCH + g * 16) + iota) << 9) | (u * mi)
            compact = _lane_gather(packed, inv)
            ibuf[b][pl.ds(cnt, 16)] = compact >> 9
            lbuf[b][pl.ds(cnt, 16)] = compact & 511

        return cnt + total

    def scan_chunk(ci, b):
        wait_dst(ci, b)

        def g_(g, c):
            return grp(ci, g, c, b)

        return lax.fori_loop(0, DCH // 16, g_, 0)

    prefetch_dst(0, 0)
    prefetch_dst(1, 1)
    cprev = scan_chunk(0, 0)
    issue_first(0)

    def outer(ci2, cprev):
        ci_a = ci2 * 2 + 1

        @pl.when(ci_a + 1 < NDCH)
        def _():
            prefetch_dst(ci_a + 1, 0)

        c_a = scan_chunk(ci_a, 1)
        issue_first(1)
        drain(0, cprev)

        @pl.when(ci_a + 2 < NDCH)
        def _():
            prefetch_dst(ci_a + 2, 1)

        c_b = scan_chunk(ci_a + 1, 0)
        issue_first(0)
        drain(1, c_a)
        return c_b

    clast = lax.fori_loop(0, (NDCH - 1) // 2, outer, cprev)
    drain(0, clast)

    rem = N - (NW - 1) * SEG

    @pl.when(w < NW - 1)
    def _():
        pltpu.sync_copy(acc.at[pl.ds(0, SEG)], o_hbm.at[pl.ds(lo, SEG)])

    @pl.when(w == NW - 1)
    def _():
        pltpu.sync_copy(acc.at[pl.ds(0, rem)], o_hbm.at[pl.ds(lo, rem)])


def _sc_scatter_max(P, dst):
    return pl.kernel(
        _scatter_body,
        mesh=_mesh(),
        out_type=jax.ShapeDtypeStruct((N, W128), jnp.float32),
        scratch_types=[
            pltpu.VMEM((DCH,), jnp.int32),
            pltpu.VMEM((DCH,), jnp.int32),
            pltpu.VMEM((IDXCAP,), jnp.int32),
            pltpu.VMEM((IDXCAP,), jnp.int32),
            pltpu.VMEM((IDXCAP + 16,), jnp.int32),
            pltpu.VMEM((IDXCAP + 16,), jnp.int32),
            pltpu.VMEM((GCH, W128), jnp.float32),
            pltpu.VMEM((GCH, W128), jnp.float32),
            pltpu.VMEM((SEG, W128), jnp.float32),
            pltpu.SemaphoreType.DMA,
            pltpu.SemaphoreType.DMA,
            pltpu.SemaphoreType.DMA,
            pltpu.SemaphoreType.DMA,
        ],
    )(P, dst)


def _prep1_body(x_ref, wsum_ref, wd_ref, ba_ref, t_ref):
    x = x_ref[...]
    t_ref[:, :H] = x @ wsum_ref[...] + ba_ref[...]
    t_ref[:, H:] = x @ wd_ref[...]


def _prep1(x, Wsum, Wd, ba):
    return pl.pallas_call(
        _prep1_body,
        out_shape=jax.ShapeDtypeStruct((N, W128), jnp.float32),
    )(x, Wsum, Wd, ba)


def _prep2_body(h_ref, x_ref, wh_ref, wd_ref, ba_ref, t_ref):
    c = x_ref[...] @ wd_ref[...]
    t_ref[:, :H] = h_ref[...][:, :H] @ wh_ref[...] + c + ba_ref[...]
    t_ref[:, H:] = c


def _prep2(h, x, Wh, Wd, ba):
    return pl.pallas_call(
        _prep2_body,
        out_shape=jax.ShapeDtypeStruct((N, W128), jnp.float32),
    )(h, x, Wh, Wd, ba)


def _edgemm_body(m_ref, wb_ref, bb_ref, p_ref):
    m = jnp.maximum(m_ref[...][:, :H], 0.0)
    p_ref[:, :H] = m @ wb_ref[...] + bb_ref[...]


def _edgemm(Mraw, Wb, bb):
    grid = (E // EBLK,)
    return pl.pallas_call(
        _edgemm_body,
        grid=grid,
        in_specs=[
            pl.BlockSpec((EBLK, W128), lambda i: (i, 0)),
            pl.BlockSpec((H, H), lambda i: (0, 0)),
            pl.BlockSpec((H,), lambda i: (0,)),
        ],
        out_specs=pl.BlockSpec((EBLK, W128), lambda i: (i, 0)),
        out_shape=jax.ShapeDtypeStruct((E, W128), jnp.float32),
    )(Mraw, Wb, bb)


def _pool_body(h_ref, batch_ref, wout_ref, bout_ref, o_ref, acc_ref):
    i = pl.program_id(0)

    @pl.when(i == 0)
    def _():
        acc_ref[...] = jnp.zeros_like(acc_ref)

    h = h_ref[...][:, :H]
    bcol = batch_ref[...]
    for b in range(B):
        mask = bcol == float(b)
        seg = jnp.max(jnp.where(mask, h, 0.0), axis=0)
        acc_ref[b, :] = jnp.maximum(acc_ref[b, :], seg)

    @pl.when(i == pl.num_programs(0) - 1)
    def _():
        o_ref[...] = acc_ref[...] @ wout_ref[...] + bout_ref[...]


def _pool(h2, batchf, Wout, bout):
    grid = (N // NBLK,)
    return pl.pallas_call(
        _pool_body,
        grid=grid,
        in_specs=[
            pl.BlockSpec((NBLK, W128), lambda i: (i, 0)),
            pl.BlockSpec((NBLK, 1), lambda i: (i, 0)),
            pl.BlockSpec((H, OUT), lambda i: (0, 0)),
            pl.BlockSpec((OUT,), lambda i: (0,)),
        ],
        out_specs=pl.BlockSpec((B, OUT), lambda i: (0, 0)),
        out_shape=jax.ShapeDtypeStruct((B, OUT), jnp.float32),
        scratch_shapes=[pltpu.VMEM((B, H), jnp.float32)],
    )(h2, batchf, Wout, bout)


def kernel(x, edge_index, batch, W1a, b1a, W1b, b1b, W2a, b2a, W2b, b2b,
           Wout, bout):
    src = edge_index[0]
    dst = edge_index[1]

    T1 = _prep1(x, W1a[:3] + W1a[3:], W1a[3:], b1a)
    M1 = _sc_gather(T1, src, dst)
    P1 = _edgemm(M1, W1b, b1b)
    h1 = _sc_scatter_max(P1, dst)

    T2 = _prep2(h1, x, W2a[:H], W2a[H:], b2a)
    M2 = _sc_gather(T2, src, dst)
    P2 = _edgemm(M2, W2b, b2b)
    h2 = _sc_scatter_max(P2, dst)

    batchf = batch.astype(jnp.float32).reshape(N, 1)
    return _pool(h2, batchf, Wout, bout)


# speedup vs baseline: 1.6720x; 1.6720x over previous
import functools

import jax
import jax.numpy as jnp
from jax import lax
from jax.experimental import pallas as pl
from jax.experimental.pallas import tpu as pltpu
from jax.experimental.pallas import tpu_sc as plsc

N = 10000
E = 320000
H = 64
W128 = 128
OUT = 128
B = 16

NCORE = 2
NSUB = 16
NW = NCORE * NSUB

EBLK = 3200
NBLK = 2000

GCH = 128
NCHUNKS = E // GCH

SEG = 320
DCH = 2560
NDCH = E // DCH
IDXCAP = GCH + 16


def _mesh():
    return plsc.VectorSubcoreMesh(core_axis_name="c", subcore_axis_name="s")


def _wid():
    return lax.axis_index("s") * NCORE + lax.axis_index("c")


def _gather_body(t_hbm, src_hbm, dst_hbm, m_hbm,
                 sbuf0, sbuf1, dbuf0, dbuf1, abuf0, abuf1, cbuf0, cbuf1,
                 mbuf, sem_a0, sem_a1, sem_c0, sem_c1):
    w = _wid()
    nch = (NCHUNKS - w + NW - 1) // NW
    sbuf = (sbuf0, sbuf1)
    dbuf = (dbuf0, dbuf1)
    abuf = (abuf0, abuf1)
    cbuf = (cbuf0, cbuf1)
    sem_a = (sem_a0, sem_a1)
    sem_c = (sem_c0, sem_c1)

    zeros = jnp.zeros((16,), jnp.float32)

    def zrow(r, _):
        for j in range(H, W128, 16):
            mbuf[r, pl.ds(j, 16)] = zeros
        return 0

    lax.fori_loop(0, GCH, zrow, 0)

    def issue(k, b):
        base = (w + k * NW) * GCH
        pltpu.sync_copy(src_hbm.at[pl.ds(base, GCH)], sbuf[b])
        pltpu.sync_copy(dst_hbm.at[pl.ds(base, GCH)], dbuf[b])
        pltpu.async_copy(t_hbm.at[sbuf[b]], abuf[b], sem_a[b])
        pltpu.async_copy(t_hbm.at[dbuf[b]], cbuf[b], sem_c[b])

    def consume(k, b):
        pltpu.make_async_copy(t_hbm.at[sbuf[b]], abuf[b], sem_a[b]).wait()
        pltpu.make_async_copy(t_hbm.at[dbuf[b]], cbuf[b], sem_c[b]).wait()

        def row(r, _):
            for j in range(0, H, 16):
                mbuf[r, pl.ds(j, 16)] = (abuf[b][r, pl.ds(j, 16)]
                                         - cbuf[b][r, pl.ds(H + j, 16)])
            return 0

        lax.fori_loop(0, GCH, row, 0, unroll=2)
        pltpu.sync_copy(mbuf, m_hbm.at[pl.ds((w + k * NW) * GCH, GCH)])

    issue(0, 0)

    def outer(k2, _):
        for b in range(2):
            k = k2 * 2 + b

            @pl.when(k + 1 < nch)
            def _():
                issue(k + 1, 1 - b)

            @pl.when(k < nch)
            def _():
                consume(k, b)
        return 0

    lax.fori_loop(0, (NCHUNKS // NW + 2) // 2, outer, 0)


def _sc_gather(T, src, dst):
    return pl.kernel(
        _gather_body,
        mesh=_mesh(),
        out_type=jax.ShapeDtypeStruct((E, W128), jnp.float32),
        scratch_types=[
            pltpu.VMEM((GCH,), jnp.int32),
            pltpu.VMEM((GCH,), jnp.int32),
            pltpu.VMEM((GCH,), jnp.int32),
            pltpu.VMEM((GCH,), jnp.int32),
            pltpu.VMEM((GCH, W128), jnp.float32),
            pltpu.VMEM((GCH, W128), jnp.float32),
            pltpu.VMEM((GCH, W128), jnp.float32),
            pltpu.VMEM((GCH, W128), jnp.float32),
            pltpu.VMEM((GCH, W128), jnp.float32),
            pltpu.SemaphoreType.DMA,
            pltpu.SemaphoreType.DMA,
            pltpu.SemaphoreType.DMA,
            pltpu.SemaphoreType.DMA,
        ],
    )(T, src, dst)


def _scatter_body(p_hbm, dst_hbm, o_hbm, dbuf0, dbuf1, ibuf, lbuf, prow,
                  acc, sem, dsem0, dsem1):
    w = _wid()
    lo = w * SEG
    dbuf = (dbuf0, dbuf1)
    dsem = (dsem0, dsem1)

    def prefetch_dst(ci, b):
        pltpu.async_copy(dst_hbm.at[pl.ds(ci * DCH, DCH)], dbuf[b], dsem[b])

    def wait_dst(ci, b):
        pltpu.make_async_copy(dst_hbm.at[pl.ds(ci * DCH, DCH)],
                              dbuf[b], dsem[b]).wait()

    zeros = jnp.zeros((16,), jnp.float32)

    def zrow(r, _):
        for j in range(0, W128, 16):
            acc[r, pl.ds(j, 16)] = zeros
        return 0

    lax.fori_loop(0, SEG, zrow, 0)

    zidx = jnp.zeros((16,), jnp.int32)

    def zi(g, _):
        ibuf[pl.ds(g * 16, 16)] = zidx
        return 0

    lax.fori_loop(0, IDXCAP // 16, zi, 0)

    def flush(cnt):
        pltpu.async_copy(p_hbm.at[ibuf], prow, sem).wait()

        def rmw(r, _):
            l = lbuf[pl.ds(r, 16)][0]
            for jj in range(0, H, 16):
                sl = pl.ds(jj, 16)
                acc[l, sl] = jnp.maximum(acc[l, sl], prow[r, sl])
            return 0

        lax.fori_loop(0, cnt, rmw, 0)
        return 0

    iota = lax.iota(jnp.int32, 16)

    def _lane_gather(v, idx):
        return v.at[idx].get(mode="promise_in_bounds")

    stepm = {s: ((iota - s) >> 31) + 1 for s in (1, 2, 4, 8)}

    def grp(ci, g, cnt, b):
        d = dbuf[b][pl.ds(g * 16, 16)]
        u = d - lo
        mi = ((u >> 31) | ((SEG - 1 - u) >> 31)) + 1
        pc = mi
        for s in (1, 2, 4, 8):
            sh = _lane_gather(pc, jnp.maximum(iota - s, 0))
            pc = pc + sh * stepm[s]
        total = pc[15]

        @pl.when(total > 0)
        def _():
            inv = jnp.zeros((16,), jnp.int32)
            for s in (8, 4, 2, 1):
                probe = _lane_gather(pc, jnp.minimum(inv + (s - 1), 15))
                inv = inv + s * (((iota - probe) >> 31) + 1)
            inv = jnp.minimum(inv, 15)
            packed = (((ci * DCH + g * 16) + iota) << 9) | (u * mi)
            compact = _lane_gather(packed, inv)
            ibuf[pl.ds(cnt, 16)] = compact >> 9
            lbuf[pl.ds(cnt, 16)] = compact & 511

        cnt = cnt + total
        cnt = lax.cond(cnt >= GCH, flush, lambda c: c, cnt)
        return cnt

    prefetch_dst(0, 0)
    prefetch_dst(1, 1)

    def outer(ci2, cnt):
        for b in range(2):
            ci = ci2 * 2 + b

            @pl.when(ci < NDCH)
            def _():
                wait_dst(ci, b)

            def g_(g, c):
                return grp(ci, g, c, b)

            cnt = lax.cond(
                ci < NDCH,
                lambda c: lax.fori_loop(0, DCH // 16, g_, c),
                lambda c: c, cnt)

            @pl.when(ci + 2 < NDCH)
            def _():
                prefetch_dst(ci + 2, b)
        return cnt

    cnt = lax.fori_loop(0, (NDCH + 1) // 2, outer, 0)
    lax.cond(cnt > 0, flush, lambda c: 0, cnt)

    rem = N - (NW - 1) * SEG

    @pl.when(w < NW - 1)
    def _():
        pltpu.sync_copy(acc.at[pl.ds(0, SEG)], o_hbm.at[pl.ds(lo, SEG)])

    @pl.when(w == NW - 1)
    def _():
        pltpu.sync_copy(acc.at[pl.ds(0, rem)], o_hbm.at[pl.ds(lo, rem)])


def _sc_scatter_max(P, dst):
    return pl.kernel(
        _scatter_body,
        mesh=_mesh(),
        out_type=jax.ShapeDtypeStruct((N, W128), jnp.float32),
        scratch_types=[
            pltpu.VMEM((DCH,), jnp.int32),
            pltpu.VMEM((DCH,), jnp.int32),
            pltpu.VMEM((IDXCAP,), jnp.int32),
            pltpu.VMEM((IDXCAP + 16,), jnp.int32),
            pltpu.VMEM((IDXCAP, W128), jnp.float32),
            pltpu.VMEM((SEG, W128), jnp.float32),
            pltpu.SemaphoreType.DMA,
            pltpu.SemaphoreType.DMA,
            pltpu.SemaphoreType.DMA,
        ],
    )(P, dst)


def _prep1_body(x_ref, wsum_ref, wd_ref, ba_ref, t_ref):
    x = x_ref[...]
    t_ref[:, :H] = x @ wsum_ref[...] + ba_ref[...]
    t_ref[:, H:] = x @ wd_ref[...]


def _prep1(x, Wsum, Wd, ba):
    return pl.pallas_call(
        _prep1_body,
        out_shape=jax.ShapeDtypeStruct((N, W128), jnp.float32),
    )(x, Wsum, Wd, ba)


def _prep2_body(h_ref, x_ref, wh_ref, wd_ref, ba_ref, t_ref):
    c = x_ref[...] @ wd_ref[...]
    t_ref[:, :H] = h_ref[...][:, :H] @ wh_ref[...] + c + ba_ref[...]
    t_ref[:, H:] = c


def _prep2(h, x, Wh, Wd, ba):
    return pl.pallas_call(
        _prep2_body,
        out_shape=jax.ShapeDtypeStruct((N, W128), jnp.float32),
    )(h, x, Wh, Wd, ba)


def _edgemm_body(m_ref, wb_ref, bb_ref, p_ref):
    m = jnp.maximum(m_ref[...][:, :H], 0.0)
    p_ref[:, :H] = m @ wb_ref[...] + bb_ref[...]


def _edgemm(Mraw, Wb, bb):
    grid = (E // EBLK,)
    return pl.pallas_call(
        _edgemm_body,
        grid=grid,
        in_specs=[
            pl.BlockSpec((EBLK, W128), lambda i: (i, 0)),
            pl.BlockSpec((H, H), lambda i: (0, 0)),
            pl.BlockSpec((H,), lambda i: (0,)),
        ],
        out_specs=pl.BlockSpec((EBLK, W128), lambda i: (i, 0)),
        out_shape=jax.ShapeDtypeStruct((E, W128), jnp.float32),
    )(Mraw, Wb, bb)


def _pool_body(h_ref, batch_ref, wout_ref, bout_ref, o_ref, acc_ref):
    i = pl.program_id(0)

    @pl.when(i == 0)
    def _():
        acc_ref[...] = jnp.zeros_like(acc_ref)

    h = h_ref[...][:, :H]
    bcol = batch_ref[...]
    for b in range(B):
        mask = bcol == float(b)
        seg = jnp.max(jnp.where(mask, h, 0.0), axis=0)
        acc_ref[b, :] = jnp.maximum(acc_ref[b, :], seg)

    @pl.when(i == pl.num_programs(0) - 1)
    def _():
        o_ref[...] = acc_ref[...] @ wout_ref[...] + bout_ref[...]


def _pool(h2, batchf, Wout, bout):
    grid = (N // NBLK,)
    return pl.pallas_call(
        _pool_body,
        grid=grid,
        in_specs=[
            pl.BlockSpec((NBLK, W128), lambda i: (i, 0)),
            pl.BlockSpec((NBLK, 1), lambda i: (i, 0)),
            pl.BlockSpec((H, OUT), lambda i: (0, 0)),
            pl.BlockSpec((OUT,), lambda i: (0,)),
        ],
        out_specs=pl.BlockSpec((B, OUT), lambda i: (0, 0)),
        out_shape=jax.ShapeDtypeStruct((B, OUT), jnp.float32),
        scratch_shapes=[pltpu.VMEM((B, H), jnp.float32)],
    )(h2, batchf, Wout, bout)


def kernel(x, edge_index, batch, W1a, b1a, W1b, b1b, W2a, b2a, W2b, b2b,
           Wout, bout):
    src = edge_index[0]
    dst = edge_index[1]

    T1 = _prep1(x, W1a[:3] + W1a[3:], W1a[3:], b1a)
    M1 = _sc_gather(T1, src, dst)
    P1 = _edgemm(M1, W1b, b1b)
    h1 = _sc_scatter_max(P1, dst)

    T2 = _prep2(h1, x, W2a[:H], W2a[H:], b2a)
    M2 = _sc_gather(T2, src, dst)
    P2 = _edgemm(M2, W2b, b2b)
    h2 = _sc_scatter_max(P2, dst)

    batchf = batch.astype(jnp.float32).reshape(N, 1)
    return _pool(h2, batchf, Wout, bout)


# speedup vs baseline: 1.7356x; 1.0381x over previous
import functools

import jax
import jax.numpy as jnp
from jax import lax
from jax.experimental import pallas as pl
from jax.experimental.pallas import tpu as pltpu
from jax.experimental.pallas import tpu_sc as plsc

N = 10000
E = 320000
H = 64
W128 = 128
OUT = 128
B = 16

NCORE = 2
NSUB = 16
NW = NCORE * NSUB

EBLK = 3200
NBLK = 2000

GCH = 128
NCHUNKS = E // GCH

SEG = 320
DCH = 2560
NDCH = E // DCH
FLUSH_AT = 384
IDXCAP = FLUSH_AT + 16


def _mesh():
    return plsc.VectorSubcoreMesh(core_axis_name="c", subcore_axis_name="s")


def _wid():
    return lax.axis_index("s") * NCORE + lax.axis_index("c")


def _gather_body(t_hbm, src_hbm, dst_hbm, m_hbm,
                 sbuf0, sbuf1, dbuf0, dbuf1, abuf0, abuf1, cbuf0, cbuf1,
                 mbuf, sem_a0, sem_a1, sem_c0, sem_c1):
    w = _wid()
    nch = (NCHUNKS - w + NW - 1) // NW
    sbuf = (sbuf0, sbuf1)
    dbuf = (dbuf0, dbuf1)
    abuf = (abuf0, abuf1)
    cbuf = (cbuf0, cbuf1)
    sem_a = (sem_a0, sem_a1)
    sem_c = (sem_c0, sem_c1)

    zeros = jnp.zeros((16,), jnp.float32)

    def zrow(r, _):
        for j in range(H, W128, 16):
            mbuf[r, pl.ds(j, 16)] = zeros
        return 0

    lax.fori_loop(0, GCH, zrow, 0)

    def issue(k, b):
        base = (w + k * NW) * GCH
        pltpu.sync_copy(src_hbm.at[pl.ds(base, GCH)], sbuf[b])
        pltpu.sync_copy(dst_hbm.at[pl.ds(base, GCH)], dbuf[b])
        pltpu.async_copy(t_hbm.at[sbuf[b]], abuf[b], sem_a[b])
        pltpu.async_copy(t_hbm.at[dbuf[b]], cbuf[b], sem_c[b])

    def consume(k, b):
        pltpu.make_async_copy(t_hbm.at[sbuf[b]], abuf[b], sem_a[b]).wait()
        pltpu.make_async_copy(t_hbm.at[dbuf[b]], cbuf[b], sem_c[b]).wait()

        def row(r, _):
            for j in range(0, H, 16):
                mbuf[r, pl.ds(j, 16)] = (abuf[b][r, pl.ds(j, 16)]
                                         - cbuf[b][r, pl.ds(H + j, 16)])
            return 0

        lax.fori_loop(0, GCH, row, 0, unroll=2)
        pltpu.sync_copy(mbuf, m_hbm.at[pl.ds((w + k * NW) * GCH, GCH)])

    issue(0, 0)

    def outer(k2, _):
        for b in range(2):
            k = k2 * 2 + b

            @pl.when(k + 1 < nch)
            def _():
                issue(k + 1, 1 - b)

            @pl.when(k < nch)
            def _():
                consume(k, b)
        return 0

    lax.fori_loop(0, (NCHUNKS // NW + 2) // 2, outer, 0)


def _sc_gather(T, src, dst):
    return pl.kernel(
        _gather_body,
        mesh=_mesh(),
        out_type=jax.ShapeDtypeStruct((E, W128), jnp.float32),
        scratch_types=[
            pltpu.VMEM((GCH,), jnp.int32),
            pltpu.VMEM((GCH,), jnp.int32),
            pltpu.VMEM((GCH,), jnp.int32),
            pltpu.VMEM((GCH,), jnp.int32),
            pltpu.VMEM((GCH, W128), jnp.float32),
            pltpu.VMEM((GCH, W128), jnp.float32),
            pltpu.VMEM((GCH, W128), jnp.float32),
            pltpu.VMEM((GCH, W128), jnp.float32),
            pltpu.VMEM((GCH, W128), jnp.float32),
            pltpu.SemaphoreType.DMA,
            pltpu.SemaphoreType.DMA,
            pltpu.SemaphoreType.DMA,
            pltpu.SemaphoreType.DMA,
        ],
    )(T, src, dst)


def _scatter_body(p_hbm, dst_hbm, o_hbm, dbuf0, dbuf1, ibuf, lbuf, prow,
                  acc, sem, dsem0, dsem1):
    w = _wid()
    lo = w * SEG
    dbuf = (dbuf0, dbuf1)
    dsem = (dsem0, dsem1)

    def prefetch_dst(ci, b):
        pltpu.async_copy(dst_hbm.at[pl.ds(ci * DCH, DCH)], dbuf[b], dsem[b])

    def wait_dst(ci, b):
        pltpu.make_async_copy(dst_hbm.at[pl.ds(ci * DCH, DCH)],
                              dbuf[b], dsem[b]).wait()

    zeros = jnp.zeros((16,), jnp.float32)

    def zrow(r, _):
        for j in range(0, W128, 16):
            acc[r, pl.ds(j, 16)] = zeros
        return 0

    lax.fori_loop(0, SEG, zrow, 0)

    zidx = jnp.zeros((16,), jnp.int32)

    def zi(g, _):
        ibuf[pl.ds(g * 16, 16)] = zidx
        return 0

    lax.fori_loop(0, IDXCAP // 16, zi, 0)

    def flush(cnt):
        pltpu.async_copy(p_hbm.at[ibuf], prow, sem).wait()

        def rmw(r, _):
            l = lbuf[pl.ds(r, 16)][0]
            for jj in range(0, H, 16):
                sl = pl.ds(jj, 16)
                acc[l, sl] = jnp.maximum(acc[l, sl], prow[r, sl])
            return 0

        lax.fori_loop(0, cnt, rmw, 0)
        return 0

    iota = lax.iota(jnp.int32, 16)

    def _lane_gather(v, idx):
        return v.at[idx].get(mode="promise_in_bounds")

    stepm = {s: ((iota - s) >> 31) + 1 for s in (1, 2, 4, 8)}

    def grp(ci, g, cnt, b):
        d = dbuf[b][pl.ds(g * 16, 16)]
        u = d - lo
        mi = ((u >> 31) | ((SEG - 1 - u) >> 31)) + 1
        pc = mi
        for s in (1, 2, 4, 8):
            sh = _lane_gather(pc, jnp.maximum(iota - s, 0))
            pc = pc + sh * stepm[s]
        total = pc[15]

        @pl.when(total > 0)
        def _():
            inv = jnp.zeros((16,), jnp.int32)
            for s in (8, 4, 2, 1):
                probe = _lane_gather(pc, jnp.minimum(inv + (s - 1), 15))
                inv = inv + s * (((iota - probe) >> 31) + 1)
            inv = jnp.minimum(inv, 15)
            packed = (((ci * DCH + g * 16) + iota) << 9) | (u * mi)
            compact = _lane_gather(packed, inv)
            ibuf[pl.ds(cnt, 16)] = compact >> 9
            lbuf[pl.ds(cnt, 16)] = compact & 511

        cnt = cnt + total
        cnt = lax.cond(cnt >= FLUSH_AT, flush, lambda c: c, cnt)
        return cnt

    prefetch_dst(0, 0)
    prefetch_dst(1, 1)

    def outer(ci2, cnt):
        for b in range(2):
            ci = ci2 * 2 + b

            @pl.when(ci < NDCH)
            def _():
                wait_dst(ci, b)

            def g_(g, c):
                return grp(ci, g, c, b)

            cnt = lax.cond(
                ci < NDCH,
                lambda c: lax.fori_loop(0, DCH // 16, g_, c),
                lambda c: c, cnt)

            @pl.when(ci + 2 < NDCH)
            def _():
                prefetch_dst(ci + 2, b)
        return cnt

    cnt = lax.fori_loop(0, (NDCH + 1) // 2, outer, 0)
    lax.cond(cnt > 0, flush, lambda c: 0, cnt)

    rem = N - (NW - 1) * SEG

    @pl.when(w < NW - 1)
    def _():
        pltpu.sync_copy(acc.at[pl.ds(0, SEG)], o_hbm.at[pl.ds(lo, SEG)])

    @pl.when(w == NW - 1)
    def _():
        pltpu.sync_copy(acc.at[pl.ds(0, rem)], o_hbm.at[pl.ds(lo, rem)])


def _sc_scatter_max(P, dst):
    return pl.kernel(
        _scatter_body,
        mesh=_mesh(),
        out_type=jax.ShapeDtypeStruct((N, W128), jnp.float32),
        scratch_types=[
            pltpu.VMEM((DCH,), jnp.int32),
            pltpu.VMEM((DCH,), jnp.int32),
            pltpu.VMEM((IDXCAP,), jnp.int32),
            pltpu.VMEM((IDXCAP + 16,), jnp.int32),
            pltpu.VMEM((IDXCAP, W128), jnp.float32),
            pltpu.VMEM((SEG, W128), jnp.float32),
            pltpu.SemaphoreType.DMA,
            pltpu.SemaphoreType.DMA,
            pltpu.SemaphoreType.DMA,
        ],
    )(P, dst)


def _prep1_body(x_ref, wsum_ref, wd_ref, ba_ref, t_ref):
    x = x_ref[...]
    t_ref[:, :H] = x @ wsum_ref[...] + ba_ref[...]
    t_ref[:, H:] = x @ wd_ref[...]


def _prep1(x, Wsum, Wd, ba):
    return pl.pallas_call(
        _prep1_body,
        out_shape=jax.ShapeDtypeStruct((N, W128), jnp.float32),
    )(x, Wsum, Wd, ba)


def _prep2_body(h_ref, x_ref, wh_ref, wd_ref, ba_ref, t_ref):
    c = x_ref[...] @ wd_ref[...]
    t_ref[:, :H] = h_ref[...][:, :H] @ wh_ref[...] + c + ba_ref[...]
    t_ref[:, H:] = c


def _prep2(h, x, Wh, Wd, ba):
    return pl.pallas_call(
        _prep2_body,
        out_shape=jax.ShapeDtypeStruct((N, W128), jnp.float32),
    )(h, x, Wh, Wd, ba)


def _edgemm_body(m_ref, wb_ref, bb_ref, p_ref):
    m = jnp.maximum(m_ref[...][:, :H], 0.0)
    p_ref[:, :H] = m @ wb_ref[...] + bb_ref[...]


def _edgemm(Mraw, Wb, bb):
    grid = (E // EBLK,)
    return pl.pallas_call(
        _edgemm_body,
        grid=grid,
        in_specs=[
            pl.BlockSpec((EBLK, W128), lambda i: (i, 0)),
            pl.BlockSpec((H, H), lambda i: (0, 0)),
            pl.BlockSpec((H,), lambda i: (0,)),
        ],
        out_specs=pl.BlockSpec((EBLK, W128), lambda i: (i, 0)),
        out_shape=jax.ShapeDtypeStruct((E, W128), jnp.float32),
    )(Mraw, Wb, bb)


def _pool_body(h_ref, batch_ref, wout_ref, bout_ref, o_ref, acc_ref):
    i = pl.program_id(0)

    @pl.when(i == 0)
    def _():
        acc_ref[...] = jnp.zeros_like(acc_ref)

    h = h_ref[...][:, :H]
    bcol = batch_ref[...]
    for b in range(B):
        mask = bcol == float(b)
        seg = jnp.max(jnp.where(mask, h, 0.0), axis=0)
        acc_ref[b, :] = jnp.maximum(acc_ref[b, :], seg)

    @pl.when(i == pl.num_programs(0) - 1)
    def _():
        o_ref[...] = acc_ref[...] @ wout_ref[...] + bout_ref[...]


def _pool(h2, batchf, Wout, bout):
    grid = (N // NBLK,)
    return pl.pallas_call(
        _pool_body,
        grid=grid,
        in_specs=[
            pl.BlockSpec((NBLK, W128), lambda i: (i, 0)),
            pl.BlockSpec((NBLK, 1), lambda i: (i, 0)),
            pl.BlockSpec((H, OUT), lambda i: (0, 0)),
            pl.BlockSpec((OUT,), lambda i: (0,)),
        ],
        out_specs=pl.BlockSpec((B, OUT), lambda i: (0, 0)),
        out_shape=jax.ShapeDtypeStruct((B, OUT), jnp.float32),
        scratch_shapes=[pltpu.VMEM((B, H), jnp.float32)],
    )(h2, batchf, Wout, bout)


def kernel(x, edge_index, batch, W1a, b1a, W1b, b1b, W2a, b2a, W2b, b2b,
           Wout, bout):
    src = edge_index[0]
    dst = edge_index[1]

    T1 = _prep1(x, W1a[:3] + W1a[3:], W1a[3:], b1a)
    M1 = _sc_gather(T1, src, dst)
    P1 = _edgemm(M1, W1b, b1b)
    h1 = _sc_scatter_max(P1, dst)

    T2 = _prep2(h1, x, W2a[:H], W2a[H:], b2a)
    M2 = _sc_gather(T2, src, dst)
    P2 = _edgemm(M2, W2b, b2b)
    h2 = _sc_scatter_max(P2, dst)

    batchf = batch.astype(jnp.float32).reshape(N, 1)
    return _pool(h2, batchf, Wout, bout)


# speedup vs baseline: 1.7422x; 1.0038x over previous
import functools

import jax
import jax.numpy as jnp
from jax import lax
from jax.experimental import pallas as pl
from jax.experimental.pallas import tpu as pltpu
from jax.experimental.pallas import tpu_sc as plsc

N = 10000
E = 320000
H = 64
W128 = 128
OUT = 128
B = 16

NCORE = 2
NSUB = 16
NW = NCORE * NSUB

EBLK = 3200
NBLK = 2000

GCH = 128
NCHUNKS = E // GCH

SEG = 320
DCH = 2560
NDCH = E // DCH
FLUSH_AT = 512
IDXCAP = FLUSH_AT + 16


def _mesh():
    return plsc.VectorSubcoreMesh(core_axis_name="c", subcore_axis_name="s")


def _wid():
    return lax.axis_index("s") * NCORE + lax.axis_index("c")


def _gather_body(t_hbm, src_hbm, dst_hbm, m_hbm,
                 sbuf0, sbuf1, dbuf0, dbuf1, abuf0, abuf1, cbuf0, cbuf1,
                 mbuf, sem_a0, sem_a1, sem_c0, sem_c1):
    w = _wid()
    nch = (NCHUNKS - w + NW - 1) // NW
    sbuf = (sbuf0, sbuf1)
    dbuf = (dbuf0, dbuf1)
    abuf = (abuf0, abuf1)
    cbuf = (cbuf0, cbuf1)
    sem_a = (sem_a0, sem_a1)
    sem_c = (sem_c0, sem_c1)

    zeros = jnp.zeros((16,), jnp.float32)

    def zrow(r, _):
        for j in range(H, W128, 16):
            mbuf[r, pl.ds(j, 16)] = zeros
        return 0

    lax.fori_loop(0, GCH, zrow, 0)

    def issue(k, b):
        base = (w + k * NW) * GCH
        pltpu.sync_copy(src_hbm.at[pl.ds(base, GCH)], sbuf[b])
        pltpu.sync_copy(dst_hbm.at[pl.ds(base, GCH)], dbuf[b])
        pltpu.async_copy(t_hbm.at[sbuf[b]], abuf[b], sem_a[b])
        pltpu.async_copy(t_hbm.at[dbuf[b]], cbuf[b], sem_c[b])

    def consume(k, b):
        pltpu.make_async_copy(t_hbm.at[sbuf[b]], abuf[b], sem_a[b]).wait()
        pltpu.make_async_copy(t_hbm.at[dbuf[b]], cbuf[b], sem_c[b]).wait()

        def row(r, _):
            for j in range(0, H, 16):
                mbuf[r, pl.ds(j, 16)] = (abuf[b][r, pl.ds(j, 16)]
                                         - cbuf[b][r, pl.ds(H + j, 16)])
            return 0

        lax.fori_loop(0, GCH, row, 0, unroll=2)
        pltpu.sync_copy(mbuf, m_hbm.at[pl.ds((w + k * NW) * GCH, GCH)])

    issue(0, 0)

    def outer(k2, _):
        for b in range(2):
            k = k2 * 2 + b

            @pl.when(k + 1 < nch)
            def _():
                issue(k + 1, 1 - b)

            @pl.when(k < nch)
            def _():
                consume(k, b)
        return 0

    lax.fori_loop(0, (NCHUNKS // NW + 2) // 2, outer, 0)


def _sc_gather(T, src, dst):
    return pl.kernel(
        _gather_body,
        mesh=_mesh(),
        out_type=jax.ShapeDtypeStruct((E, W128), jnp.float32),
        scratch_types=[
            pltpu.VMEM((GCH,), jnp.int32),
            pltpu.VMEM((GCH,), jnp.int32),
            pltpu.VMEM((GCH,), jnp.int32),
            pltpu.VMEM((GCH,), jnp.int32),
            pltpu.VMEM((GCH, W128), jnp.float32),
            pltpu.VMEM((GCH, W128), jnp.float32),
            pltpu.VMEM((GCH, W128), jnp.float32),
            pltpu.VMEM((GCH, W128), jnp.float32),
            pltpu.VMEM((GCH, W128), jnp.float32),
            pltpu.SemaphoreType.DMA,
            pltpu.SemaphoreType.DMA,
            pltpu.SemaphoreType.DMA,
            pltpu.SemaphoreType.DMA,
        ],
    )(T, src, dst)


def _scatter_body(p_hbm, dst_hbm, o_hbm, dbuf0, dbuf1, ibuf, lbuf, prow,
                  acc, sem, dsem0, dsem1):
    w = _wid()
    lo = w * SEG
    dbuf = (dbuf0, dbuf1)
    dsem = (dsem0, dsem1)

    def prefetch_dst(ci, b):
        pltpu.async_copy(dst_hbm.at[pl.ds(ci * DCH, DCH)], dbuf[b], dsem[b])

    def wait_dst(ci, b):
        pltpu.make_async_copy(dst_hbm.at[pl.ds(ci * DCH, DCH)],
                              dbuf[b], dsem[b]).wait()

    zeros = jnp.zeros((16,), jnp.float32)

    def zrow(r, _):
        for j in range(0, W128, 16):
            acc[r, pl.ds(j, 16)] = zeros
        return 0

    lax.fori_loop(0, SEG, zrow, 0)

    zidx = jnp.zeros((16,), jnp.int32)

    def zi(g, _):
        ibuf[pl.ds(g * 16, 16)] = zidx
        return 0

    lax.fori_loop(0, IDXCAP // 16, zi, 0)

    def flush(cnt):
        pltpu.async_copy(p_hbm.at[ibuf], prow, sem).wait()

        def rmw(r, _):
            l = lbuf[pl.ds(r, 16)][0]
            for jj in range(0, H, 16):
                sl = pl.ds(jj, 16)
                acc[l, sl] = jnp.maximum(acc[l, sl], prow[r, sl])
            return 0

        lax.fori_loop(0, cnt, rmw, 0)
        return 0

    iota = lax.iota(jnp.int32, 16)

    def _lane_gather(v, idx):
        return v.at[idx].get(mode="promise_in_bounds")

    stepm = {s: ((iota - s) >> 31) + 1 for s in (1, 2, 4, 8)}

    def grp(ci, g, cnt, b):
        d = dbuf[b][pl.ds(g * 16, 16)]
        u = d - lo
        mi = ((u >> 31) | ((SEG - 1 - u) >> 31)) + 1
        pc = mi
        for s in (1, 2, 4, 8):
            sh = _lane_gather(pc, jnp.maximum(iota - s, 0))
            pc = pc + sh * stepm[s]
        total = pc[15]

        @pl.when(total > 0)
        def _():
            inv = jnp.zeros((16,), jnp.int32)
            for s in (8, 4, 2, 1):
                probe = _lane_gather(pc, jnp.minimum(inv + (s - 1), 15))
                inv = inv + s * (((iota - probe) >> 31) + 1)
            inv = jnp.minimum(inv, 15)
            packed = (((ci * DCH + g * 16) + iota) << 9) | (u * mi)
            compact = _lane_gather(packed, inv)
            ibuf[pl.ds(cnt, 16)] = compact >> 9
            lbuf[pl.ds(cnt, 16)] = compact & 511

        cnt = cnt + total
        cnt = lax.cond(cnt >= FLUSH_AT, flush, lambda c: c, cnt)
        return cnt

    prefetch_dst(0, 0)
    prefetch_dst(1, 1)

    def outer(ci2, cnt):
        for b in range(2):
            ci = ci2 * 2 + b

            @pl.when(ci < NDCH)
            def _():
                wait_dst(ci, b)

            def g_(g, c):
                return grp(ci, g, c, b)

            cnt = lax.cond(
                ci < NDCH,
                lambda c: lax.fori_loop(0, DCH // 16, g_, c),
                lambda c: c, cnt)

            @pl.when(ci + 2 < NDCH)
            def _():
                prefetch_dst(ci + 2, b)
        return cnt

    cnt = lax.fori_loop(0, (NDCH + 1) // 2, outer, 0)
    lax.cond(cnt > 0, flush, lambda c: 0, cnt)

    rem = N - (NW - 1) * SEG

    @pl.when(w < NW - 1)
    def _():
        pltpu.sync_copy(acc.at[pl.ds(0, SEG)], o_hbm.at[pl.ds(lo, SEG)])

    @pl.when(w == NW - 1)
    def _():
        pltpu.sync_copy(acc.at[pl.ds(0, rem)], o_hbm.at[pl.ds(lo, rem)])


def _sc_scatter_max(P, dst):
    return pl.kernel(
        _scatter_body,
        mesh=_mesh(),
        out_type=jax.ShapeDtypeStruct((N, W128), jnp.float32),
        scratch_types=[
            pltpu.VMEM((DCH,), jnp.int32),
            pltpu.VMEM((DCH,), jnp.int32),
            pltpu.VMEM((IDXCAP,), jnp.int32),
            pltpu.VMEM((IDXCAP + 16,), jnp.int32),
            pltpu.VMEM((IDXCAP, W128), jnp.float32),
            pltpu.VMEM((SEG, W128), jnp.float32),
            pltpu.SemaphoreType.DMA,
            pltpu.SemaphoreType.DMA,
            pltpu.SemaphoreType.DMA,
        ],
    )(P, dst)


def _prep1_body(x_ref, wsum_ref, wd_ref, ba_ref, t_ref):
    x = x_ref[...]
    t_ref[:, :H] = x @ wsum_ref[...] + ba_ref[...]
    t_ref[:, H:] = x @ wd_ref[...]


def _prep1(x, Wsum, Wd, ba):
    return pl.pallas_call(
        _prep1_body,
        out_shape=jax.ShapeDtypeStruct((N, W128), jnp.float32),
    )(x, Wsum, Wd, ba)


def _prep2_body(h_ref, x_ref, wh_ref, wd_ref, ba_ref, t_ref):
    c = x_ref[...] @ wd_ref[...]
    t_ref[:, :H] = h_ref[...][:, :H] @ wh_ref[...] + c + ba_ref[...]
    t_ref[:, H:] = c


def _prep2(h, x, Wh, Wd, ba):
    return pl.pallas_call(
        _prep2_body,
        out_shape=jax.ShapeDtypeStruct((N, W128), jnp.float32),
    )(h, x, Wh, Wd, ba)


def _edgemm_body(m_ref, wb_ref, bb_ref, p_ref):
    m = jnp.maximum(m_ref[...][:, :H], 0.0)
    p_ref[:, :H] = m @ wb_ref[...] + bb_ref[...]


def _edgemm(Mraw, Wb, bb):
    grid = (E // EBLK,)
    return pl.pallas_call(
        _edgemm_body,
        grid=grid,
        in_specs=[
            pl.BlockSpec((EBLK, W128), lambda i: (i, 0)),
            pl.BlockSpec((H, H), lambda i: (0, 0)),
            pl.BlockSpec((H,), lambda i: (0,)),
        ],
        out_specs=pl.BlockSpec((EBLK, W128), lambda i: (i, 0)),
        out_shape=jax.ShapeDtypeStruct((E, W128), jnp.float32),
    )(Mraw, Wb, bb)


def _pool_body(h_ref, batch_ref, wout_ref, bout_ref, o_ref, acc_ref):
    i = pl.program_id(0)

    @pl.when(i == 0)
    def _():
        acc_ref[...] = jnp.zeros_like(acc_ref)

    h = h_ref[...][:, :H]
    bcol = batch_ref[...]
    for b in range(B):
        mask = bcol == float(b)
        seg = jnp.max(jnp.where(mask, h, 0.0), axis=0)
        acc_ref[b, :] = jnp.maximum(acc_ref[b, :], seg)

    @pl.when(i == pl.num_programs(0) - 1)
    def _():
        o_ref[...] = acc_ref[...] @ wout_ref[...] + bout_ref[...]


def _pool(h2, batchf, Wout, bout):
    grid = (N // NBLK,)
    return pl.pallas_call(
        _pool_body,
        grid=grid,
        in_specs=[
            pl.BlockSpec((NBLK, W128), lambda i: (i, 0)),
            pl.BlockSpec((NBLK, 1), lambda i: (i, 0)),
            pl.BlockSpec((H, OUT), lambda i: (0, 0)),
            pl.BlockSpec((OUT,), lambda i: (0,)),
        ],
        out_specs=pl.BlockSpec((B, OUT), lambda i: (0, 0)),
        out_shape=jax.ShapeDtypeStruct((B, OUT), jnp.float32),
        scratch_shapes=[pltpu.VMEM((B, H), jnp.float32)],
    )(h2, batchf, Wout, bout)


def kernel(x, edge_index, batch, W1a, b1a, W1b, b1b, W2a, b2a, W2b, b2b,
           Wout, bout):
    src = edge_index[0]
    dst = edge_index[1]

    T1 = _prep1(x, W1a[:3] + W1a[3:], W1a[3:], b1a)
    M1 = _sc_gather(T1, src, dst)
    P1 = _edgemm(M1, W1b, b1b)
    h1 = _sc_scatter_max(P1, dst)

    T2 = _prep2(h1, x, W2a[:H], W2a[H:], b2a)
    M2 = _sc_gather(T2, src, dst)
    P2 = _edgemm(M2, W2b, b2b)
    h2 = _sc_scatter_max(P2, dst)

    batchf = batch.astype(jnp.float32).reshape(N, 1)
    return _pool(h2, batchf, Wout, bout)


# speedup vs baseline: 1.7432x; 1.0006x over previous
import functools

import jax
import jax.numpy as jnp
from jax import lax
from jax.experimental import pallas as pl
from jax.experimental.pallas import tpu as pltpu
from jax.experimental.pallas import tpu_sc as plsc

N = 10000
E = 320000
H = 64
W128 = 128
OUT = 128
B = 16

NCORE = 2
NSUB = 16
NW = NCORE * NSUB

EBLK = 3200
NBLK = 2000

GCH = 128
NCHUNKS = E // GCH

SEG = 320
DCH = 4000
NDCH = E // DCH
FLUSH_AT = 512
IDXCAP = FLUSH_AT + 16


def _mesh():
    return plsc.VectorSubcoreMesh(core_axis_name="c", subcore_axis_name="s")


def _wid():
    return lax.axis_index("s") * NCORE + lax.axis_index("c")


def _gather_body(t_hbm, src_hbm, dst_hbm, m_hbm,
                 sbuf0, sbuf1, dbuf0, dbuf1, abuf0, abuf1, cbuf0, cbuf1,
                 mbuf, sem_a0, sem_a1, sem_c0, sem_c1):
    w = _wid()
    nch = (NCHUNKS - w + NW - 1) // NW
    sbuf = (sbuf0, sbuf1)
    dbuf = (dbuf0, dbuf1)
    abuf = (abuf0, abuf1)
    cbuf = (cbuf0, cbuf1)
    sem_a = (sem_a0, sem_a1)
    sem_c = (sem_c0, sem_c1)

    zeros = jnp.zeros((16,), jnp.float32)

    def zrow(r, _):
        for j in range(H, W128, 16):
            mbuf[r, pl.ds(j, 16)] = zeros
        return 0

    lax.fori_loop(0, GCH, zrow, 0)

    def issue(k, b):
        base = (w + k * NW) * GCH
        pltpu.sync_copy(src_hbm.at[pl.ds(base, GCH)], sbuf[b])
        pltpu.sync_copy(dst_hbm.at[pl.ds(base, GCH)], dbuf[b])
        pltpu.async_copy(t_hbm.at[sbuf[b]], abuf[b], sem_a[b])
        pltpu.async_copy(t_hbm.at[dbuf[b]], cbuf[b], sem_c[b])

    def consume(k, b):
        pltpu.make_async_copy(t_hbm.at[sbuf[b]], abuf[b], sem_a[b]).wait()
        pltpu.make_async_copy(t_hbm.at[dbuf[b]], cbuf[b], sem_c[b]).wait()

        def row(r, _):
            for j in range(0, H, 16):
                mbuf[r, pl.ds(j, 16)] = (abuf[b][r, pl.ds(j, 16)]
                                         - cbuf[b][r, pl.ds(H + j, 16)])
            return 0

        lax.fori_loop(0, GCH, row, 0, unroll=2)
        pltpu.sync_copy(mbuf, m_hbm.at[pl.ds((w + k * NW) * GCH, GCH)])

    issue(0, 0)

    def outer(k2, _):
        for b in range(2):
            k = k2 * 2 + b

            @pl.when(k + 1 < nch)
            def _():
                issue(k + 1, 1 - b)

            @pl.when(k < nch)
            def _():
                consume(k, b)
        return 0

    lax.fori_loop(0, (NCHUNKS // NW + 2) // 2, outer, 0)


def _sc_gather(T, src, dst):
    return pl.kernel(
        _gather_body,
        mesh=_mesh(),
        out_type=jax.ShapeDtypeStruct((E, W128), jnp.float32),
        scratch_types=[
            pltpu.VMEM((GCH,), jnp.int32),
            pltpu.VMEM((GCH,), jnp.int32),
            pltpu.VMEM((GCH,), jnp.int32),
            pltpu.VMEM((GCH,), jnp.int32),
            pltpu.VMEM((GCH, W128), jnp.float32),
            pltpu.VMEM((GCH, W128), jnp.float32),
            pltpu.VMEM((GCH, W128), jnp.float32),
            pltpu.VMEM((GCH, W128), jnp.float32),
            pltpu.VMEM((GCH, W128), jnp.float32),
            pltpu.SemaphoreType.DMA,
            pltpu.SemaphoreType.DMA,
            pltpu.SemaphoreType.DMA,
            pltpu.SemaphoreType.DMA,
        ],
    )(T, src, dst)


def _scatter_body(p_hbm, dst_hbm, o_hbm, dbuf0, dbuf1, ibuf, lbuf, prow,
                  acc, sem, dsem0, dsem1):
    w = _wid()
    lo = w * SEG
    dbuf = (dbuf0, dbuf1)
    dsem = (dsem0, dsem1)

    def prefetch_dst(ci, b):
        pltpu.async_copy(dst_hbm.at[pl.ds(ci * DCH, DCH)], dbuf[b], dsem[b])

    def wait_dst(ci, b):
        pltpu.make_async_copy(dst_hbm.at[pl.ds(ci * DCH, DCH)],
                              dbuf[b], dsem[b]).wait()

    zeros = jnp.zeros((16,), jnp.float32)

    def zrow(r, _):
        for j in range(0, W128, 16):
            acc[r, pl.ds(j, 16)] = zeros
        return 0

    lax.fori_loop(0, SEG, zrow, 0)

    zidx = jnp.zeros((16,), jnp.int32)

    def zi(g, _):
        ibuf[pl.ds(g * 16, 16)] = zidx
        return 0

    lax.fori_loop(0, IDXCAP // 16, zi, 0)

    def flush(cnt):
        pltpu.async_copy(p_hbm.at[ibuf], prow, sem).wait()

        def rmw(r, _):
            l = lbuf[pl.ds(r, 16)][0]
            for jj in range(0, H, 16):
                sl = pl.ds(jj, 16)
                acc[l, sl] = jnp.maximum(acc[l, sl], prow[r, sl])
            return 0

        lax.fori_loop(0, cnt, rmw, 0)
        return 0

    iota = lax.iota(jnp.int32, 16)

    def _lane_gather(v, idx):
        return v.at[idx].get(mode="promise_in_bounds")

    stepm = {s: ((iota - s) >> 31) + 1 for s in (1, 2, 4, 8)}

    def grp(ci, g, cnt, b):
        d = dbuf[b][pl.ds(g * 16, 16)]
        u = d - lo
        mi = ((u >> 31) | ((SEG - 1 - u) >> 31)) + 1
        pc = mi
        for s in (1, 2, 4, 8):
            sh = _lane_gather(pc, jnp.maximum(iota - s, 0))
            pc = pc + sh * stepm[s]
        total = pc[15]

        @pl.when(total > 0)
        def _():
            inv = jnp.zeros((16,), jnp.int32)
            for s in (8, 4, 2, 1):
                probe = _lane_gather(pc, jnp.minimum(inv + (s - 1), 15))
                inv = inv + s * (((iota - probe) >> 31) + 1)
            inv = jnp.minimum(inv, 15)
            packed = (((ci * DCH + g * 16) + iota) << 9) | (u * mi)
            compact = _lane_gather(packed, inv)
            ibuf[pl.ds(cnt, 16)] = compact >> 9
            lbuf[pl.ds(cnt, 16)] = compact & 511

        cnt = cnt + total
        cnt = lax.cond(cnt >= FLUSH_AT, flush, lambda c: c, cnt)
        return cnt

    prefetch_dst(0, 0)
    prefetch_dst(1, 1)

    def outer(ci2, cnt):
        for b in range(2):
            ci = ci2 * 2 + b

            @pl.when(ci < NDCH)
            def _():
                wait_dst(ci, b)

            def g_(g, c):
                return grp(ci, g, c, b)

            cnt = lax.cond(
                ci < NDCH,
                lambda c: lax.fori_loop(0, DCH // 16, g_, c),
                lambda c: c, cnt)

            @pl.when(ci + 2 < NDCH)
            def _():
                prefetch_dst(ci + 2, b)
        return cnt

    cnt = lax.fori_loop(0, (NDCH + 1) // 2, outer, 0)
    lax.cond(cnt > 0, flush, lambda c: 0, cnt)

    rem = N - (NW - 1) * SEG

    @pl.when(w < NW - 1)
    def _():
        pltpu.sync_copy(acc.at[pl.ds(0, SEG)], o_hbm.at[pl.ds(lo, SEG)])

    @pl.when(w == NW - 1)
    def _():
        pltpu.sync_copy(acc.at[pl.ds(0, rem)], o_hbm.at[pl.ds(lo, rem)])


def _sc_scatter_max(P, dst):
    return pl.kernel(
        _scatter_body,
        mesh=_mesh(),
        out_type=jax.ShapeDtypeStruct((N, W128), jnp.float32),
        scratch_types=[
            pltpu.VMEM((DCH,), jnp.int32),
            pltpu.VMEM((DCH,), jnp.int32),
            pltpu.VMEM((IDXCAP,), jnp.int32),
            pltpu.VMEM((IDXCAP + 16,), jnp.int32),
            pltpu.VMEM((IDXCAP, W128), jnp.float32),
            pltpu.VMEM((SEG, W128), jnp.float32),
            pltpu.SemaphoreType.DMA,
            pltpu.SemaphoreType.DMA,
            pltpu.SemaphoreType.DMA,
        ],
    )(P, dst)


def _prep1_body(x_ref, wsum_ref, wd_ref, ba_ref, t_ref):
    x = x_ref[...]
    t_ref[:, :H] = x @ wsum_ref[...] + ba_ref[...]
    t_ref[:, H:] = x @ wd_ref[...]


def _prep1(x, Wsum, Wd, ba):
    return pl.pallas_call(
        _prep1_body,
        out_shape=jax.ShapeDtypeStruct((N, W128), jnp.float32),
    )(x, Wsum, Wd, ba)


def _prep2_body(h_ref, x_ref, wh_ref, wd_ref, ba_ref, t_ref):
    c = x_ref[...] @ wd_ref[...]
    t_ref[:, :H] = h_ref[...][:, :H] @ wh_ref[...] + c + ba_ref[...]
    t_ref[:, H:] = c


def _prep2(h, x, Wh, Wd, ba):
    return pl.pallas_call(
        _prep2_body,
        out_shape=jax.ShapeDtypeStruct((N, W128), jnp.float32),
    )(h, x, Wh, Wd, ba)


def _edgemm_body(m_ref, wb_ref, bb_ref, p_ref):
    m = jnp.maximum(m_ref[...][:, :H], 0.0)
    p_ref[:, :H] = m @ wb_ref[...] + bb_ref[...]


def _edgemm(Mraw, Wb, bb):
    grid = (E // EBLK,)
    return pl.pallas_call(
        _edgemm_body,
        grid=grid,
        in_specs=[
            pl.BlockSpec((EBLK, W128), lambda i: (i, 0)),
            pl.BlockSpec((H, H), lambda i: (0, 0)),
            pl.BlockSpec((H,), lambda i: (0,)),
        ],
        out_specs=pl.BlockSpec((EBLK, W128), lambda i: (i, 0)),
        out_shape=jax.ShapeDtypeStruct((E, W128), jnp.float32),
    )(Mraw, Wb, bb)


def _pool_body(h_ref, batch_ref, wout_ref, bout_ref, o_ref, acc_ref):
    i = pl.program_id(0)

    @pl.when(i == 0)
    def _():
        acc_ref[...] = jnp.zeros_like(acc_ref)

    h = h_ref[...][:, :H]
    bcol = batch_ref[...]
    for b in range(B):
        mask = bcol == float(b)
        seg = jnp.max(jnp.where(mask, h, 0.0), axis=0)
        acc_ref[b, :] = jnp.maximum(acc_ref[b, :], seg)

    @pl.when(i == pl.num_programs(0) - 1)
    def _():
        o_ref[...] = acc_ref[...] @ wout_ref[...] + bout_ref[...]


def _pool(h2, batchf, Wout, bout):
    grid = (N // NBLK,)
    return pl.pallas_call(
        _pool_body,
        grid=grid,
        in_specs=[
            pl.BlockSpec((NBLK, W128), lambda i: (i, 0)),
            pl.BlockSpec((NBLK, 1), lambda i: (i, 0)),
            pl.BlockSpec((H, OUT), lambda i: (0, 0)),
            pl.BlockSpec((OUT,), lambda i: (0,)),
        ],
        out_specs=pl.BlockSpec((B, OUT), lambda i: (0, 0)),
        out_shape=jax.ShapeDtypeStruct((B, OUT), jnp.float32),
        scratch_shapes=[pltpu.VMEM((B, H), jnp.float32)],
    )(h2, batchf, Wout, bout)


def kernel(x, edge_index, batch, W1a, b1a, W1b, b1b, W2a, b2a, W2b, b2b,
           Wout, bout):
    src = edge_index[0]
    dst = edge_index[1]

    T1 = _prep1(x, W1a[:3] + W1a[3:], W1a[3:], b1a)
    M1 = _sc_gather(T1, src, dst)
    P1 = _edgemm(M1, W1b, b1b)
    h1 = _sc_scatter_max(P1, dst)

    T2 = _prep2(h1, x, W2a[:H], W2a[H:], b2a)
    M2 = _sc_gather(T2, src, dst)
    P2 = _edgemm(M2, W2b, b2b)
    h2 = _sc_scatter_max(P2, dst)

    batchf = batch.astype(jnp.float32).reshape(N, 1)
    return _pool(h2, batchf, Wout, bout)
